# Initial kernel scaffold; baseline (speedup 1.0000x reference)
import jax
import jax.numpy as jnp
from jax.experimental import pallas as pl


def kernel(pos_seq, pe_k_weight):
    raise NotImplementedError("write your pallas kernel here")


# speedup vs baseline: 4.0028x; 4.0028x over previous
import functools

import jax
import jax.numpy as jnp
from jax import lax
from jax.experimental import pallas as pl
from jax.experimental.pallas import tpu as pltpu
from jax.experimental.pallas import tpu_sc as plsc

_MAXLEN = 8000
_D = 1024
_BATCH = 4
_POS_LEN = 16383
_WPB = 8
_RPW = 2048
_K = 32
_NSTEPS = _RPW // _K
_NBUF = 3
_LAST_BASE = 16376 - _RPW
_TAIL_BASE = _POS_LEN - 16
_IDX_MAIN = _BATCH * _WPB * _RPW
_IDX_LEN = _IDX_MAIN + _BATCH * 16


def _sc_body(idx_hbm, table_hbm, out_hbm,
             idx_v, oidx_v, r0, r1, r2, gs0, gs1, gs2, ws0, ws1, ws2):
    rows = (r0, r1, r2)
    gsem = (gs0, gs1, gs2)
    wsem = (ws0, ws1, ws2)

    c = lax.axis_index("c")
    s = lax.axis_index("s")
    wid = s * 2 + c
    batch = wid // _WPB
    j = wid % _WPB
    base_out = pl.multiple_of(jnp.minimum(j * _RPW, _LAST_BASE), 8)

    pltpu.sync_copy(idx_hbm.at[pl.ds(pl.multiple_of(wid * _RPW, 8), _RPW)],
                    idx_v)

    @pl.loop(0, _RPW, step=16)
    def _clamp(i):
        v = idx_v[pl.ds(i, 16)]
        idx_v[pl.ds(i, 16)] = jnp.clip(v, -_MAXLEN, _MAXLEN - 1) + _MAXLEN

    def start_gather(g, b):
        pltpu.async_copy(table_hbm.at[idx_v.at[pl.ds(g * _K, _K)]],
                         rows[b], gsem[b])

    def wait_gather(b):
        pltpu.make_async_copy(table_hbm.at[idx_v.at[pl.ds(0, _K)]],
                              rows[b], gsem[b]).wait()

    def start_write(g, b):
        start = pl.multiple_of(base_out + g * _K, 8)
        pltpu.async_copy(rows[b], out_hbm.at[batch, pl.ds(start, _K)],
                         wsem[b])

    def wait_write(b):
        pltpu.make_async_copy(rows[b], out_hbm.at[batch, pl.ds(base_out, _K)],
                              wsem[b]).wait()

    start_gather(0, 0)
    wait_gather(0); start_write(0, 0); start_gather(1, 1)
    wait_gather(1); start_write(1, 1); start_gather(2, 2)
    wait_gather(2); start_write(2, 2); wait_write(0); start_gather(3, 0)

    @pl.loop(3, _NSTEPS - 1, step=_NBUF)
    def _steady(g0):
        for db in range(_NBUF):
            g = g0 + db
            b = db
            bn = (db + 1) % _NBUF
            wait_gather(b)
            start_write(g, b)
            wait_write(bn)
            start_gather(g + 1, bn)

    wait_gather(0)
    start_write(_NSTEPS - 1, 0)
    wait_write(0); wait_write(1); wait_write(2)

    @pl.when(j == _WPB - 1)
    def _tail():
        toff = pl.multiple_of(_IDX_MAIN + batch * 16, 8)
        pltpu.sync_copy(idx_hbm.at[pl.ds(toff, 16)], idx_v.at[pl.ds(0, 16)])
        v = idx_v[pl.ds(0, 16)]
        idx_v[pl.ds(0, 16)] = jnp.clip(v, -_MAXLEN, _MAXLEN - 1) + _MAXLEN
        oidx_v[...] = lax.iota(jnp.int32, 16) + _TAIL_BASE
        pltpu.async_copy(table_hbm.at[idx_v.at[pl.ds(0, 16)]],
                         r0.at[pl.ds(0, 16)], gs0).wait()
        pltpu.async_copy(r0.at[pl.ds(0, 16)],
                         out_hbm.at[batch].at[oidx_v], ws0).wait()


_mesh = plsc.VectorSubcoreMesh(core_axis_name="c", subcore_axis_name="s")

_gather_call = functools.partial(
    pl.kernel,
    out_type=jax.ShapeDtypeStruct((_BATCH, _POS_LEN, _D), jnp.float32),
    mesh=_mesh,
    scratch_types=[
        pltpu.VMEM((_RPW,), jnp.int32),
        pltpu.VMEM((16,), jnp.int32),
        pltpu.VMEM((_K, _D), jnp.float32),
        pltpu.VMEM((_K, _D), jnp.float32),
        pltpu.VMEM((_K, _D), jnp.float32),
        pltpu.SemaphoreType.DMA,
        pltpu.SemaphoreType.DMA,
        pltpu.SemaphoreType.DMA,
        pltpu.SemaphoreType.DMA,
        pltpu.SemaphoreType.DMA,
        pltpu.SemaphoreType.DMA,
    ],
)(_sc_body)


@jax.jit
def kernel(pos_seq, pe_k_weight):
    pos = pos_seq.astype(jnp.int32)
    parts = []
    for b in range(_BATCH):
        parts.append(pos[b, : (_WPB - 1) * _RPW])
        parts.append(pos[b, _LAST_BASE:_LAST_BASE + _RPW])
    for b in range(_BATCH):
        parts.append(pos[b, _TAIL_BASE:])
    idx_arr = jnp.concatenate(parts)
    out = _gather_call(idx_arr, pe_k_weight)
    return out, None


# speedup vs baseline: 7.9291x; 1.9809x over previous
import functools

import jax
import jax.numpy as jnp
from jax import lax
from jax.experimental import pallas as pl
from jax.experimental.pallas import tpu as pltpu
from jax.experimental.pallas import tpu_sc as plsc

_MAXLEN = 8000
_D = 1024
_BATCH = 4
_POS_LEN = 16383
_NW = 32
_PPW = 512
_RPW = _BATCH * _PPW
_LAST_P0 = _POS_LEN - _PPW
_NBUF = 4


def _sc_body(idx_hbm, table_hbm, out_hbm,
             idx_v, a0, a1, a2, a3, gs0, gs1, gs2, gs3, ws0, ws1, ws2, ws3):
    bufs = (a0, a1, a2, a3)
    gsem = (gs0, gs1, gs2, gs3)
    wsem = (ws0, ws1, ws2, ws3)

    c = lax.axis_index("c")
    s = lax.axis_index("s")
    wid = s * 2 + c
    p0 = jnp.minimum(wid * _PPW, _LAST_P0)

    pltpu.sync_copy(idx_hbm.at[pl.ds(pl.multiple_of(p0 * 8, 8), _PPW * 8)],
                    idx_v)

    @pl.loop(0, _PPW * 8, step=16)
    def _clamp(i):
        v = idx_v[pl.ds(i, 16)]
        idx_v[pl.ds(i, 16)] = jnp.clip(v, -_MAXLEN, _MAXLEN - 1) + _MAXLEN

    def start_gather(p, b):
        pltpu.async_copy(table_hbm.at[idx_v.at[pl.ds(p * 8, _BATCH)]],
                         bufs[b], gsem[b])

    def wait_gather(b):
        pltpu.make_async_copy(table_hbm.at[idx_v.at[pl.ds(0, _BATCH)]],
                              bufs[b], gsem[b]).wait()

    def start_write(p, b):
        pltpu.async_copy(bufs[b], out_hbm.at[p0 + p], wsem[b])

    def wait_write(b):
        pltpu.make_async_copy(bufs[b], out_hbm.at[p0], wsem[b]).wait()

    start_gather(0, 0); start_gather(1, 1); start_gather(2, 2)
    wait_gather(0); start_write(0, 0); start_gather(3, 3)
    wait_gather(1); start_write(1, 1); wait_write(0); start_gather(4, 0)
    wait_gather(2); start_write(2, 2); wait_write(1); start_gather(5, 1)
    wait_gather(3); start_write(3, 3); wait_write(2); start_gather(6, 2)

    @pl.loop(4, _PPW - 4, step=_NBUF)
    def _steady(pg):
        for db in range(_NBUF):
            p = pg + db
            b = db
            bn = (db + 3) % _NBUF
            wait_gather(b)
            start_write(p, b)
            wait_write(bn)
            start_gather(p + 3, bn)

    wait_gather(0); start_write(_PPW - 4, 0)
    wait_write(3); start_gather(_PPW - 1, 3)
    wait_gather(1); start_write(_PPW - 3, 1)
    wait_gather(2); start_write(_PPW - 2, 2)
    wait_gather(3); start_write(_PPW - 1, 3)
    wait_write(0); wait_write(1); wait_write(2); wait_write(3)


_mesh = plsc.VectorSubcoreMesh(core_axis_name="c", subcore_axis_name="s")

_gather_call = functools.partial(
    pl.kernel,
    out_type=jax.ShapeDtypeStruct((_POS_LEN, _BATCH, _D), jnp.float32),
    mesh=_mesh,
    scratch_types=[
        pltpu.VMEM((_PPW * 8,), jnp.int32),
        pltpu.VMEM((_BATCH, _D), jnp.float32),
        pltpu.VMEM((_BATCH, _D), jnp.float32),
        pltpu.VMEM((_BATCH, _D), jnp.float32),
        pltpu.VMEM((_BATCH, _D), jnp.float32),
        pltpu.SemaphoreType.DMA,
        pltpu.SemaphoreType.DMA,
        pltpu.SemaphoreType.DMA,
        pltpu.SemaphoreType.DMA,
        pltpu.SemaphoreType.DMA,
        pltpu.SemaphoreType.DMA,
        pltpu.SemaphoreType.DMA,
        pltpu.SemaphoreType.DMA,
    ],
)(_sc_body)


@jax.jit
def kernel(pos_seq, pe_k_weight):
    idx_t = pos_seq.astype(jnp.int32).T
    idx2 = jnp.concatenate([idx_t, jnp.zeros_like(idx_t)], axis=1)
    idx2 = jnp.pad(idx2, ((0, 1), (0, 0)))
    out3 = _gather_call(idx2.reshape(-1), pe_k_weight)
    return jnp.transpose(out3, (1, 0, 2)), None


# speedup vs baseline: 8.8244x; 1.1129x over previous
import functools

import jax
import jax.numpy as jnp
from jax import lax
from jax.experimental import pallas as pl
from jax.experimental.pallas import tpu as pltpu
from jax.experimental.pallas import tpu_sc as plsc

_MAXLEN = 8000
_D = 1024
_BATCH = 4
_POS_LEN = 16383
_NW = 32
_PPW = 512
_LAST_P0 = _POS_LEN - _PPW
_KP = 8
_NSTEPS = _PPW // _KP
_NBUF = 3


def _sc_body(idx_hbm, table_hbm, out_hbm,
             idx_v, w0, w1, w2, gs0, gs1, gs2, ws0, ws1, ws2):
    bufs = (w0, w1, w2)
    gsem = (gs0, gs1, gs2)
    wsem = (ws0, ws1, ws2)

    c = lax.axis_index("c")
    s = lax.axis_index("s")
    wid = s * 2 + c
    p0 = jnp.minimum(wid * _PPW, _LAST_P0)

    pltpu.sync_copy(idx_hbm.at[pl.ds(pl.multiple_of(p0 * 8, 8), _PPW * 8)],
                    idx_v)

    @pl.loop(0, _PPW * 8, step=16)
    def _clamp(i):
        v = idx_v[pl.ds(i, 16)]
        idx_v[pl.ds(i, 16)] = jnp.clip(v, -_MAXLEN, _MAXLEN - 1) + _MAXLEN

    def fire_gathers(sc, b):
        for di in range(_KP):
            pltpu.async_copy(
                table_hbm.at[idx_v.at[pl.ds((sc * _KP + di) * 8, _BATCH)]],
                bufs[b].at[di], gsem[b])

    def wait_gathers(b):
        pltpu.make_async_copy(out_hbm.at[pl.ds(0, _KP)], bufs[b],
                              gsem[b]).wait()

    def start_write(sc, b):
        pltpu.async_copy(bufs[b], out_hbm.at[pl.ds(p0 + sc * _KP, _KP)],
                         wsem[b])

    def wait_write(b):
        pltpu.make_async_copy(bufs[b], out_hbm.at[pl.ds(p0, _KP)],
                              wsem[b]).wait()

    fire_gathers(0, 0)
    wait_gathers(0); start_write(0, 0); fire_gathers(1, 1)
    wait_gathers(1); start_write(1, 1); fire_gathers(2, 2)
    wait_gathers(2); start_write(2, 2); wait_write(0); fire_gathers(3, 0)

    @pl.loop(3, _NSTEPS - 1, step=_NBUF)
    def _steady(s0):
        for db in range(_NBUF):
            sc = s0 + db
            b = db
            bn = (db + 1) % _NBUF
            wait_gathers(b)
            start_write(sc, b)
            wait_write(bn)
            fire_gathers(sc + 1, bn)

    wait_gathers(0)
    start_write(_NSTEPS - 1, 0)
    wait_write(0); wait_write(1); wait_write(2)


_mesh = plsc.VectorSubcoreMesh(core_axis_name="c", subcore_axis_name="s")

_gather_call = functools.partial(
    pl.kernel,
    out_type=jax.ShapeDtypeStruct((_POS_LEN, _BATCH, _D), jnp.float32),
    mesh=_mesh,
    scratch_types=[
        pltpu.VMEM((_PPW * 8,), jnp.int32),
        pltpu.VMEM((_KP, _BATCH, _D), jnp.float32),
        pltpu.VMEM((_KP, _BATCH, _D), jnp.float32),
        pltpu.VMEM((_KP, _BATCH, _D), jnp.float32),
        pltpu.SemaphoreType.DMA,
        pltpu.SemaphoreType.DMA,
        pltpu.SemaphoreType.DMA,
        pltpu.SemaphoreType.DMA,
        pltpu.SemaphoreType.DMA,
        pltpu.SemaphoreType.DMA,
    ],
)(_sc_body)


@jax.jit
def kernel(pos_seq, pe_k_weight):
    idx_t = pos_seq.astype(jnp.int32).T
    idx2 = jnp.pad(idx_t, ((0, 1), (0, 4)))
    out3 = _gather_call(idx2.reshape(-1), pe_k_weight)
    return jnp.transpose(out3, (1, 0, 2)), None


# speedup vs baseline: 8.9660x; 1.0160x over previous
import functools

import jax
import jax.numpy as jnp
from jax import lax
from jax.experimental import pallas as pl
from jax.experimental.pallas import tpu as pltpu
from jax.experimental.pallas import tpu_sc as plsc

_MAXLEN = 8000
_D = 1024
_BATCH = 4
_POS_LEN = 16383
_NW = 32
_PPW = 512
_KP = 8
_NSTEPS = _PPW // _KP
_NBUF = 3


def _sc_body(idx_hbm, table_hbm, out_hbm,
             i0, i1, i2, w0, w1, w2,
             is0, is1, is2, gs0, gs1, gs2, ws0, ws1, ws2):
    ibufs = (i0, i1, i2)
    bufs = (w0, w1, w2)
    isem = (is0, is1, is2)
    gsem = (gs0, gs1, gs2)
    wsem = (ws0, ws1, ws2)

    c = lax.axis_index("c")
    s = lax.axis_index("s")
    wid = s * 2 + c
    p0 = wid * _PPW

    def il(sc, b):
        pltpu.async_copy(
            idx_hbm.at[pl.ds(pl.multiple_of(p0 + sc * _KP, 8), _KP), :],
            ibufs[b], isem[b])

    def wait_idx(b):
        pltpu.make_async_copy(idx_hbm.at[pl.ds(0, _KP), :], ibufs[b],
                              isem[b]).wait()

    def fire_gathers(sc, b):
        del sc
        for di in range(_KP):
            pltpu.async_copy(
                table_hbm.at[ibufs[b].at[di, pl.ds(0, _BATCH)]],
                bufs[b].at[di], gsem[b])

    def wait_gathers(b):
        pltpu.make_async_copy(out_hbm.at[pl.ds(0, _KP)], bufs[b],
                              gsem[b]).wait()

    def start_write(sc, b):
        pltpu.async_copy(bufs[b], out_hbm.at[pl.ds(p0 + sc * _KP, _KP)],
                         wsem[b])

    def wait_write(b):
        pltpu.make_async_copy(bufs[b], out_hbm.at[pl.ds(0, _KP)],
                              wsem[b]).wait()

    il(0, 0); il(1, 1); il(2, 2)
    wait_idx(0); fire_gathers(0, 0)
    wait_gathers(0); start_write(0, 0); il(3, 0)
    wait_idx(1); fire_gathers(1, 1)
    wait_gathers(1); start_write(1, 1); il(4, 1)
    wait_idx(2); fire_gathers(2, 2)
    wait_gathers(2); start_write(2, 2); il(5, 2)
    wait_idx(0); wait_write(0); fire_gathers(3, 0)

    @pl.loop(3, _NSTEPS - 4, step=_NBUF)
    def _steady(s0):
        for db in range(_NBUF):
            sc = s0 + db
            b = db
            bn = (db + 1) % _NBUF
            wait_gathers(b)
            start_write(sc, b)
            il(sc + 3, b)
            wait_idx(bn)
            wait_write(bn)
            fire_gathers(sc + 1, bn)

    wait_gathers(0); start_write(60, 0); il(63, 0)
    wait_idx(1); wait_write(1); fire_gathers(61, 1)
    wait_gathers(1); start_write(61, 1)
    wait_idx(2); wait_write(2); fire_gathers(62, 2)
    wait_gathers(2); start_write(62, 2)
    wait_idx(0); wait_write(0); fire_gathers(63, 0)

    wait_gathers(0)

    @pl.when(wid < _NW - 1)
    def _full():
        start_write(63, 0)
        wait_write(0)

    @pl.when(wid == _NW - 1)
    def _trim():
        pltpu.async_copy(bufs[0].at[pl.ds(0, _KP - 1)],
                         out_hbm.at[pl.ds(p0 + 63 * _KP, _KP - 1)], wsem[0])
        pltpu.make_async_copy(bufs[0].at[pl.ds(0, _KP - 1)],
                              out_hbm.at[pl.ds(0, _KP - 1)], wsem[0]).wait()

    wait_write(1); wait_write(2)


_mesh = plsc.VectorSubcoreMesh(core_axis_name="c", subcore_axis_name="s")

_gather_call = functools.partial(
    pl.kernel,
    out_type=jax.ShapeDtypeStruct((_POS_LEN, _BATCH, _D), jnp.float32),
    mesh=_mesh,
    scratch_types=[
        pltpu.VMEM((_KP, _BATCH), jnp.int32),
        pltpu.VMEM((_KP, _BATCH), jnp.int32),
        pltpu.VMEM((_KP, _BATCH), jnp.int32),
        pltpu.VMEM((_KP, _BATCH, _D), jnp.float32),
        pltpu.VMEM((_KP, _BATCH, _D), jnp.float32),
        pltpu.VMEM((_KP, _BATCH, _D), jnp.float32),
        pltpu.SemaphoreType.DMA,
        pltpu.SemaphoreType.DMA,
        pltpu.SemaphoreType.DMA,
        pltpu.SemaphoreType.DMA,
        pltpu.SemaphoreType.DMA,
        pltpu.SemaphoreType.DMA,
        pltpu.SemaphoreType.DMA,
        pltpu.SemaphoreType.DMA,
        pltpu.SemaphoreType.DMA,
    ],
)(_sc_body)


@jax.jit
def kernel(pos_seq, pe_k_weight):
    idx = jnp.clip(pos_seq.astype(jnp.int32), -_MAXLEN, _MAXLEN - 1) + _MAXLEN
    idx2d = jnp.pad(idx, ((0, 0), (0, 1))).T
    out3 = _gather_call(idx2d, pe_k_weight)
    return jnp.transpose(out3, (1, 0, 2)), None


# speedup vs baseline: 9.0063x; 1.0045x over previous
import functools

import jax
import jax.numpy as jnp
from jax import lax
from jax.experimental import pallas as pl
from jax.experimental.pallas import tpu as pltpu
from jax.experimental.pallas import tpu_sc as plsc

_MAXLEN = 8000
_D = 1024
_BATCH = 4
_POS_LEN = 16383
_NW = 32
_PPW = 512
_KP = 8
_NSTEPS = _PPW // _KP
_NBUF = 3


def _sc_body(idx_hbm, table_hbm, out_hbm,
             i0, i1, i2, w0, w1, w2,
             is0, is1, is2, gs0, gs1, gs2, ws0, ws1, ws2):
    ibufs = (i0, i1, i2)
    bufs = (w0, w1, w2)
    isem = (is0, is1, is2)
    gsem = (gs0, gs1, gs2)
    wsem = (ws0, ws1, ws2)

    c = lax.axis_index("c")
    s = lax.axis_index("s")
    wid = s * 2 + c
    p0 = wid * _PPW

    def il(sc, b):
        pltpu.async_copy(
            idx_hbm.at[pl.ds(pl.multiple_of(p0 + sc * _KP, 8), _KP), :],
            ibufs[b], isem[b])

    def wait_idx(b):
        pltpu.make_async_copy(idx_hbm.at[pl.ds(0, _KP), :], ibufs[b],
                              isem[b]).wait()

    def fire_gathers(sc, b):
        del sc
        for di in range(_KP):
            pltpu.async_copy(
                table_hbm.at[ibufs[b].at[di, pl.ds(0, _BATCH)]],
                bufs[b].at[di], gsem[b])

    def wait_gathers(b):
        pltpu.make_async_copy(out_hbm.at[pl.ds(0, _KP)], bufs[b],
                              gsem[b]).wait()

    def start_write(sc, b):
        pltpu.async_copy(bufs[b], out_hbm.at[pl.ds(p0 + sc * _KP, _KP)],
                         wsem[b])

    def wait_write(b):
        pltpu.make_async_copy(bufs[b], out_hbm.at[pl.ds(0, _KP)],
                              wsem[b]).wait()

    il(0, 0); il(1, 1); il(2, 2)
    wait_idx(0); fire_gathers(0, 0)
    wait_gathers(0); start_write(0, 0); il(3, 0)
    wait_idx(1); fire_gathers(1, 1)
    wait_gathers(1); start_write(1, 1); il(4, 1)
    wait_idx(2); fire_gathers(2, 2)
    wait_gathers(2); start_write(2, 2); il(5, 2)
    wait_idx(0); wait_write(0); fire_gathers(3, 0)

    @pl.loop(3, _NSTEPS - 4, step=_NBUF)
    def _steady(s0):
        for db in range(_NBUF):
            sc = s0 + db
            b = db
            bn = (db + 1) % _NBUF
            wait_gathers(b)
            start_write(sc, b)
            il(sc + 3, b)
            wait_idx(bn)
            wait_write(bn)
            fire_gathers(sc + 1, bn)

    wait_gathers(0); start_write(60, 0); il(63, 0)
    wait_idx(1); wait_write(1); fire_gathers(61, 1)
    wait_gathers(1); start_write(61, 1)
    wait_idx(2); wait_write(2); fire_gathers(62, 2)
    wait_gathers(2); start_write(62, 2)
    wait_idx(0); wait_write(0); fire_gathers(63, 0)

    wait_gathers(0)

    @pl.when(wid < _NW - 1)
    def _full():
        start_write(63, 0)
        wait_write(0)

    @pl.when(wid == _NW - 1)
    def _trim():
        pltpu.async_copy(bufs[0].at[pl.ds(0, _KP - 1)],
                         out_hbm.at[pl.ds(p0 + 63 * _KP, _KP - 1)], wsem[0])
        pltpu.make_async_copy(bufs[0].at[pl.ds(0, _KP - 1)],
                              out_hbm.at[pl.ds(0, _KP - 1)], wsem[0]).wait()

    wait_write(1); wait_write(2)


_mesh = plsc.VectorSubcoreMesh(core_axis_name="c", subcore_axis_name="s")

_gather_call = functools.partial(
    pl.kernel,
    out_type=jax.ShapeDtypeStruct((_POS_LEN, _BATCH, _D), jnp.float32),
    mesh=_mesh,
    scratch_types=[
        pltpu.VMEM((_KP, _BATCH), jnp.int32),
        pltpu.VMEM((_KP, _BATCH), jnp.int32),
        pltpu.VMEM((_KP, _BATCH), jnp.int32),
        pltpu.VMEM((_KP, _BATCH, _D), jnp.float32),
        pltpu.VMEM((_KP, _BATCH, _D), jnp.float32),
        pltpu.VMEM((_KP, _BATCH, _D), jnp.float32),
        pltpu.SemaphoreType.DMA,
        pltpu.SemaphoreType.DMA,
        pltpu.SemaphoreType.DMA,
        pltpu.SemaphoreType.DMA,
        pltpu.SemaphoreType.DMA,
        pltpu.SemaphoreType.DMA,
        pltpu.SemaphoreType.DMA,
        pltpu.SemaphoreType.DMA,
        pltpu.SemaphoreType.DMA,
    ],
)(_sc_body)


@jax.jit
def kernel(pos_seq, pe_k_weight):
    padded = jnp.pad(pos_seq.astype(jnp.int32), ((0, 0), (0, 1)))
    idx2d = (jnp.clip(padded, -_MAXLEN, _MAXLEN - 1) + _MAXLEN).T
    out3 = _gather_call(idx2d, pe_k_weight)
    return jnp.transpose(out3, (1, 0, 2)), None
